# Initial kernel scaffold; baseline (speedup 1.0000x reference)
import jax
import jax.numpy as jnp
from jax.experimental import pallas as pl


def kernel(x, W_router, b_router, W1, b1, W2, b2):
    raise NotImplementedError("write your pallas kernel here")


# speedup vs baseline: 1.0274x; 1.0274x over previous
import functools

import jax
import jax.numpy as jnp
from jax.experimental import pallas as pl
from jax.experimental.pallas import tpu as pltpu

_B, _S, _H = 2, 2048, 1024
_F = 4096
_E = 8
_K = 2
_COEF = 0.01
_T = _B * _S
_FC = 512


def _gelu_tanh(x):
    c = jnp.sqrt(2.0 / jnp.pi).astype(x.dtype)
    return 0.5 * x * (1.0 + jnp.tanh(c * (x + 0.044715 * (x * x * x))))


def _router_kernel(x_ref, wr_ref, br_ref, wfullt_ref, aux_ref):
    x = x_ref[...]
    wr = wr_ref[...]
    logits = jnp.dot(x, wr, preferred_element_type=jnp.float32) + br_ref[...]
    p = jax.nn.softmax(logits, axis=-1)
    e_iota = jax.lax.broadcasted_iota(jnp.int32, p.shape, 1)
    a1 = jnp.argmax(p, axis=-1)
    is1 = e_iota == a1[:, None]
    m1 = jnp.max(p, axis=-1)
    p_mask = jnp.where(is1, -jnp.inf, p)
    a2 = jnp.argmax(p_mask, axis=-1)
    m2 = jnp.max(p_mask, axis=-1)
    s = m1 + m2
    et = jax.lax.broadcasted_iota(jnp.int32, (_E, _T), 0)
    is1t = et == a1[None, :]
    is2t = et == a2[None, :]
    wfullt = (jnp.where(is1t, (m1 / s)[None, :], 0.0)
              + jnp.where(is2t, (m2 / s)[None, :], 0.0))
    wfullt_ref[...] = wfullt.astype(jnp.float32)
    cnt = jnp.sum(is1t.astype(jnp.float32), axis=1) + jnp.sum(
        is2t.astype(jnp.float32), axis=1)
    f = cnt / jnp.float32(_T * _K)
    pm = jnp.mean(p, axis=0)
    aux = jnp.float32(_E * _COEF) * jnp.sum(f * pm)
    aux_ref[...] = jnp.broadcast_to(aux, (1, 1))


def _ffn_kernel(xbf_ref, wf_ref, w1_ref, b1_ref, w2_ref, b2_ref, out_ref):
    e = pl.program_id(0)
    fc = pl.program_id(1)

    @pl.when((e == 0) & (fc == 0))
    def _():
        out_ref[...] = jnp.zeros_like(out_ref)

    xbf = xbf_ref[...]
    w1 = w1_ref[0].astype(jnp.bfloat16)
    h = jnp.dot(xbf, w1, preferred_element_type=jnp.float32) + b1_ref[0]
    h = _gelu_tanh(h)
    w2 = w2_ref[0].astype(jnp.bfloat16)
    part = jnp.dot(h.astype(jnp.bfloat16), w2,
                   preferred_element_type=jnp.float32)
    we = wf_ref[0]

    @pl.when(fc == 0)
    def _():
        out_ref[...] += we * b2_ref[0]

    out_ref[...] += part * we


def kernel(x, W_router, b_router, W1, b1, W2, b2):
    x_flat = x.reshape(_T, _H)
    wfullt, aux = pl.pallas_call(
        _router_kernel,
        out_shape=(
            jax.ShapeDtypeStruct((_E, _T), jnp.float32),
            jax.ShapeDtypeStruct((1, 1), jnp.float32),
        ),
        in_specs=[
            pl.BlockSpec((_T, _H), lambda: (0, 0)),
            pl.BlockSpec((_H, _E), lambda: (0, 0)),
            pl.BlockSpec((1, _E), lambda: (0, 0)),
        ],
        out_specs=(
            pl.BlockSpec((_E, _T), lambda: (0, 0)),
            pl.BlockSpec((1, 1), lambda: (0, 0)),
        ),
    )(x_flat, W_router, b_router.reshape(1, _E))

    xbf = x_flat.astype(jnp.bfloat16)
    wf3 = wfullt.reshape(_E, _T, 1)
    b1r = b1.reshape(_E, 1, _F)
    b2r = b2.reshape(_E, 1, _H)
    n_fc = _F // _FC
    out = pl.pallas_call(
        _ffn_kernel,
        grid=(_E, n_fc),
        out_shape=jax.ShapeDtypeStruct((_T, _H), jnp.float32),
        in_specs=[
            pl.BlockSpec((_T, _H), lambda e, fc: (0, 0)),
            pl.BlockSpec((1, _T, 1), lambda e, fc: (e, 0, 0)),
            pl.BlockSpec((1, _H, _FC), lambda e, fc: (e, 0, fc)),
            pl.BlockSpec((1, 1, _FC), lambda e, fc: (e, 0, fc)),
            pl.BlockSpec((1, _FC, _H), lambda e, fc: (e, fc, 0)),
            pl.BlockSpec((1, 1, _H), lambda e, fc: (e, 0, 0)),
        ],
        out_specs=pl.BlockSpec((_T, _H), lambda e, fc: (0, 0)),
        compiler_params=pltpu.CompilerParams(
            dimension_semantics=("arbitrary", "arbitrary"),
        ),
    )(xbf, wf3, W1, b1r, W2, b2r)

    return out.reshape(_B, _S, _H), aux.reshape(())


# speedup vs baseline: 1.5187x; 1.4782x over previous
import functools

import jax
import jax.numpy as jnp
from jax import lax
from jax.experimental import pallas as pl
from jax.experimental.pallas import tpu as pltpu
from jax.experimental.pallas import tpu_sc as plsc

_B, _S, _H = 2, 2048, 1024
_F = 4096
_E = 8
_K = 2
_COEF = 0.01
_T = _B * _S
_BLK = 256
_NB = 40
_NPAD = _NB * _BLK
_CH = 512
_NC, _NS = 2, 16
_NW = _NC * _NS
_TPW = _T // _NW
_TPS = _T // _NS
_SPW = _NPAD // _NW
_GCH = 64
_CCH = 32

_mesh = plsc.VectorSubcoreMesh(core_axis_name="c", subcore_axis_name="s",
                               num_cores=_NC, num_subcores=_NS)


def _gelu_tanh(x):
    c = jnp.sqrt(2.0 / jnp.pi).astype(x.dtype)
    return 0.5 * x * (1.0 + jnp.tanh(c * (x + 0.044715 * (x * x * x))))


def _router_meta_kernel(x_ref, wr_ref, br_ref,
                        w1_ref, w2_ref, pos1_ref, pos2_ref, be_ref, aux_ref):
    x = x_ref[...]
    wr = wr_ref[...]
    logits = jnp.dot(x, wr, preferred_element_type=jnp.float32) + br_ref[...]
    p = jax.nn.softmax(logits, axis=-1)
    e_iota = jax.lax.broadcasted_iota(jnp.int32, p.shape, 1)
    a1 = jnp.argmax(p, axis=-1)
    is1 = e_iota == a1[:, None]
    m1 = jnp.max(p, axis=-1, keepdims=True)
    p_mask = jnp.where(is1, -jnp.inf, p)
    a2 = jnp.argmax(p_mask, axis=-1)
    is2 = e_iota == a2[:, None]
    m2 = jnp.max(p_mask, axis=-1, keepdims=True)
    s = m1 + m2
    w1_ref[...] = m1 / s
    w2_ref[...] = m2 / s

    assign = (is1 | is2).astype(jnp.float32)
    cnt = jnp.sum(assign, axis=0, keepdims=True)
    padded = jnp.floor((cnt + jnp.float32(_BLK - 1)) / _BLK) * _BLK
    i8r = jax.lax.broadcasted_iota(jnp.int32, (_E, _E), 0)
    i8c = jax.lax.broadcasted_iota(jnp.int32, (_E, _E), 1)
    su8 = (i8r < i8c).astype(jnp.float32)
    po = jnp.dot(padded, su8, preferred_element_type=jnp.float32)

    po_next = po + padded
    b_iota = jax.lax.broadcasted_iota(
        jnp.int32, (64, 1), 0).astype(jnp.float32) * _BLK
    ccmp = (po_next <= b_iota).astype(jnp.int32)
    be_ref[...] = jnp.minimum(
        jnp.sum(ccmp, axis=1, keepdims=True), _E - 1)

    f = cnt / jnp.float32(_T * _K)
    pm = jnp.mean(p, axis=0, keepdims=True)
    aux = jnp.float32(_E * _COEF) * jnp.sum(f * pm)
    aux_ref[...] = jnp.broadcast_to(aux, (1, 1))

    ic_r = jax.lax.broadcasted_iota(jnp.int32, (_CH, _CH), 0)
    ic_c = jax.lax.broadcasted_iota(jnp.int32, (_CH, _CH), 1)
    sl = (ic_c < ic_r).astype(jnp.float32)
    run = jnp.zeros((1, _E), jnp.float32)
    for c in range(_T // _CH):
        lo, hi = c * _CH, (c + 1) * _CH
        ch = assign[lo:hi, :]
        ex = jnp.dot(sl, ch, preferred_element_type=jnp.float32) + run
        run = run + jnp.sum(ch, axis=0, keepdims=True)
        base = ex + po
        is1_c = is1[lo:hi, :].astype(jnp.float32)
        is2_c = is2[lo:hi, :].astype(jnp.float32)
        pos1_ref[lo:hi, :] = jnp.sum(base * is1_c, axis=1,
                                     keepdims=True).astype(jnp.int32)
        pos2_ref[lo:hi, :] = jnp.sum(base * is2_c, axis=1,
                                     keepdims=True).astype(jnp.int32)


def _router_meta(x_flat, W_router, b_router):
    return pl.pallas_call(
        _router_meta_kernel,
        out_shape=(
            jax.ShapeDtypeStruct((_T, 1), jnp.float32),
            jax.ShapeDtypeStruct((_T, 1), jnp.float32),
            jax.ShapeDtypeStruct((_T, 1), jnp.int32),
            jax.ShapeDtypeStruct((_T, 1), jnp.int32),
            jax.ShapeDtypeStruct((64, 1), jnp.int32),
            jax.ShapeDtypeStruct((1, 1), jnp.float32),
        ),
    )(x_flat, W_router, b_router.reshape(1, _E))


def _sc_dispatch_body(x_hbm, pos1_hbm, pos2_hbm, wa_hbm, wb_hbm,
                      xs_hbm, ws_hbm,
                      inv_sh, ws_sh,
                      idx1_v, idx2_v, tok_v, wa_v, wb_v,
                      zi_v, zf_v, invv, wsv, row_v, sem):
    cid = lax.axis_index("c")
    sid = lax.axis_index("s")
    wid = cid * _NS + sid

    stripe0 = sid * (_NPAD // _NS)
    for i in range((_NPAD // _NS) // 16):
        zi_v[pl.ds(i * 16, 16)] = jnp.zeros((16,), jnp.int32)
        zf_v[pl.ds(i * 16, 16)] = jnp.zeros((16,), jnp.float32)
    pltpu.sync_copy(zi_v, inv_sh.at[pl.ds(stripe0, _NPAD // _NS)])
    pltpu.sync_copy(zf_v, ws_sh.at[pl.ds(stripe0, _NPAD // _NS)])
    plsc.subcore_barrier()

    tok0 = sid * _TPS
    pltpu.sync_copy(pos1_hbm.at[pl.ds(tok0, _TPS)], idx1_v)
    pltpu.sync_copy(pos2_hbm.at[pl.ds(tok0, _TPS)], idx2_v)
    pltpu.sync_copy(wa_hbm.at[pl.ds(tok0, _TPS)], wa_v)
    pltpu.sync_copy(wb_hbm.at[pl.ds(tok0, _TPS)], wb_v)
    for i in range(_TPS // 16):
        tok_v[pl.ds(i * 16, 16)] = lax.iota(jnp.int32, 16) + (tok0 + i * 16)
    pltpu.sync_copy(tok_v, inv_sh.at[idx1_v])
    pltpu.sync_copy(tok_v, inv_sh.at[idx2_v])
    pltpu.sync_copy(wa_v, ws_sh.at[idx1_v])
    pltpu.sync_copy(wb_v, ws_sh.at[idx2_v])
    plsc.subcore_barrier()

    slot0 = wid * _SPW
    pltpu.sync_copy(inv_sh.at[pl.ds(slot0, _SPW)], invv)
    pltpu.sync_copy(ws_sh.at[pl.ds(slot0, _SPW)], wsv)
    pltpu.sync_copy(wsv, ws_hbm.at[pl.ds(slot0, _SPW)])
    for c in range(_SPW // _GCH):
        pltpu.async_copy(
            x_hbm.at[invv.at[pl.ds(c * _GCH, _GCH)]], row_v, sem).wait()
        pltpu.sync_copy(row_v, xs_hbm.at[pl.ds(slot0 + c * _GCH, _GCH)])


def _sc_dispatch(x_flat, pos1, pos2, wa, wb):
    return pl.kernel(
        _sc_dispatch_body,
        out_type=(
            jax.ShapeDtypeStruct((_NPAD, _H), jnp.float32),
            jax.ShapeDtypeStruct((_NPAD,), jnp.float32),
        ),
        mesh=_mesh,
        scratch_types=[
            pltpu.VMEM_SHARED((_NPAD,), jnp.int32),
            pltpu.VMEM_SHARED((_NPAD,), jnp.float32),
            pltpu.VMEM((_TPS,), jnp.int32),
            pltpu.VMEM((_TPS,), jnp.int32),
            pltpu.VMEM((_TPS,), jnp.int32),
            pltpu.VMEM((_TPS,), jnp.float32),
            pltpu.VMEM((_TPS,), jnp.float32),
            pltpu.VMEM((_NPAD // _NS,), jnp.int32),
            pltpu.VMEM((_NPAD // _NS,), jnp.float32),
            pltpu.VMEM((_SPW,), jnp.int32),
            pltpu.VMEM((_SPW,), jnp.float32),
            pltpu.VMEM((_GCH, _H), jnp.float32),
            pltpu.SemaphoreType.DMA,
        ],
    )(x_flat, pos1, pos2, wa, wb)


def _up_kernel(be_ref, xs_ref, w1_ref, b1_ref, h_ref):
    xb = xs_ref[...].astype(jnp.bfloat16)
    w1 = w1_ref[0].astype(jnp.bfloat16)
    h = jnp.dot(xb, w1, preferred_element_type=jnp.float32) + b1_ref[0]
    h_ref[...] = _gelu_tanh(h).astype(jnp.bfloat16)


def _down_kernel(be_ref, h_ref, w2_ref, b2_ref, ws_ref, y_ref):
    w2 = w2_ref[0].astype(jnp.bfloat16)
    y = jnp.dot(h_ref[...], w2, preferred_element_type=jnp.float32) + b2_ref[0]
    y_ref[...] = y * ws_ref[...]


def _ffn_grouped(be40, xs, W1, b1, W2, b2, ws2d):
    b1r = b1.reshape(_E, 1, _F)
    b2r = b2.reshape(_E, 1, _H)
    h = pl.pallas_call(
        _up_kernel,
        grid_spec=pltpu.PrefetchScalarGridSpec(
            num_scalar_prefetch=1,
            grid=(_NB,),
            in_specs=[
                pl.BlockSpec((_BLK, _H), lambda b, be: (b, 0)),
                pl.BlockSpec((1, _H, _F), lambda b, be: (be[b], 0, 0)),
                pl.BlockSpec((1, 1, _F), lambda b, be: (be[b], 0, 0)),
            ],
            out_specs=pl.BlockSpec((_BLK, _F), lambda b, be: (b, 0)),
        ),
        out_shape=jax.ShapeDtypeStruct((_NPAD, _F), jnp.bfloat16),
        compiler_params=pltpu.CompilerParams(
            dimension_semantics=("arbitrary",)),
    )(be40, xs, W1, b1r)
    y = pl.pallas_call(
        _down_kernel,
        grid_spec=pltpu.PrefetchScalarGridSpec(
            num_scalar_prefetch=1,
            grid=(_NB,),
            in_specs=[
                pl.BlockSpec((_BLK, _F), lambda b, be: (b, 0)),
                pl.BlockSpec((1, _F, _H), lambda b, be: (be[b], 0, 0)),
                pl.BlockSpec((1, 1, _H), lambda b, be: (be[b], 0, 0)),
                pl.BlockSpec((_BLK, 1), lambda b, be: (b, 0)),
            ],
            out_specs=pl.BlockSpec((_BLK, _H), lambda b, be: (b, 0)),
        ),
        out_shape=jax.ShapeDtypeStruct((_NPAD, _H), jnp.float32),
        compiler_params=pltpu.CompilerParams(
            dimension_semantics=("arbitrary",)),
    )(be40, h, W2, b2r, ws2d)
    return y


def _sc_combine_body(y_hbm, pos1_hbm, pos2_hbm, out_hbm,
                     idx1_v, idx2_v, b1_v, b2_v, sem1, sem2):
    cid = lax.axis_index("c")
    sid = lax.axis_index("s")
    wid = cid * _NS + sid
    tok0 = wid * _TPW
    pltpu.sync_copy(pos1_hbm.at[pl.ds(tok0, _TPW)], idx1_v)
    pltpu.sync_copy(pos2_hbm.at[pl.ds(tok0, _TPW)], idx2_v)
    for c in range(_TPW // _CCH):
        d1 = pltpu.async_copy(
            y_hbm.at[idx1_v.at[pl.ds(c * _CCH, _CCH)]], b1_v, sem1)
        d2 = pltpu.async_copy(
            y_hbm.at[idx2_v.at[pl.ds(c * _CCH, _CCH)]], b2_v, sem2)
        d1.wait()
        d2.wait()

        @pl.loop(0, _CCH)
        def _(i):
            for v in range(_H // 16):
                sl = pl.ds(v * 16, 16)
                b1_v[i, sl] = b1_v[i, sl] + b2_v[i, sl]

        pltpu.sync_copy(b1_v, out_hbm.at[pl.ds(tok0 + c * _CCH, _CCH)])


def _sc_combine(y, pos1, pos2):
    return pl.kernel(
        _sc_combine_body,
        out_type=jax.ShapeDtypeStruct((_T, _H), jnp.float32),
        mesh=_mesh,
        scratch_types=[
            pltpu.VMEM((_TPW,), jnp.int32),
            pltpu.VMEM((_TPW,), jnp.int32),
            pltpu.VMEM((_CCH, _H), jnp.float32),
            pltpu.VMEM((_CCH, _H), jnp.float32),
            pltpu.SemaphoreType.DMA,
            pltpu.SemaphoreType.DMA,
        ],
    )(y, pos1, pos2)


def kernel(x, W_router, b_router, W1, b1, W2, b2):
    x_flat = x.reshape(_T, _H)
    w1c, w2c, pos1c, pos2c, bec, aux = _router_meta(x_flat, W_router, b_router)
    pos1 = pos1c.reshape(_T)
    pos2 = pos2c.reshape(_T)
    be40 = bec.reshape(64)[:_NB]
    xs, ws = _sc_dispatch(x_flat, pos1, pos2,
                          w1c.reshape(_T), w2c.reshape(_T))
    y = _ffn_grouped(be40, xs, W1, b1, W2, b2, ws.reshape(_NPAD, 1))
    out = _sc_combine(y, pos1, pos2)
    return out.reshape(_B, _S, _H), aux.reshape(())


# speedup vs baseline: 1.5270x; 1.0055x over previous
import functools

import jax
import jax.numpy as jnp
from jax import lax
from jax.experimental import pallas as pl
from jax.experimental.pallas import tpu as pltpu
from jax.experimental.pallas import tpu_sc as plsc

_B, _S, _H = 2, 2048, 1024
_F = 4096
_E = 8
_K = 2
_COEF = 0.01
_T = _B * _S
_BLK = 256
_NB = 40
_NPAD = _NB * _BLK
_CH = 512
_NC, _NS = 2, 16
_NW = _NC * _NS
_TPW = _T // _NW
_TPS = _T // _NS
_SPW = _NPAD // _NW
_GCH = 40
_CCH = 32

_mesh = plsc.VectorSubcoreMesh(core_axis_name="c", subcore_axis_name="s",
                               num_cores=_NC, num_subcores=_NS)


def _gelu_tanh(x):
    c = jnp.sqrt(2.0 / jnp.pi).astype(x.dtype)
    return 0.5 * x * (1.0 + jnp.tanh(c * (x + 0.044715 * (x * x * x))))


def _router_meta_kernel(x_ref, wr_ref, br_ref,
                        w1_ref, w2_ref, pos1_ref, pos2_ref, be_ref, aux_ref):
    x = x_ref[...]
    wr = wr_ref[...]
    logits = jnp.dot(x, wr, preferred_element_type=jnp.float32) + br_ref[...]
    p = jax.nn.softmax(logits, axis=-1)
    e_iota = jax.lax.broadcasted_iota(jnp.int32, p.shape, 1)
    a1 = jnp.argmax(p, axis=-1)
    is1 = e_iota == a1[:, None]
    m1 = jnp.max(p, axis=-1, keepdims=True)
    p_mask = jnp.where(is1, -jnp.inf, p)
    a2 = jnp.argmax(p_mask, axis=-1)
    is2 = e_iota == a2[:, None]
    m2 = jnp.max(p_mask, axis=-1, keepdims=True)
    s = m1 + m2
    w1_ref[...] = m1 / s
    w2_ref[...] = m2 / s

    assign = (is1 | is2).astype(jnp.float32)
    cnt = jnp.sum(assign, axis=0, keepdims=True)
    padded = jnp.floor((cnt + jnp.float32(_BLK - 1)) / _BLK) * _BLK
    i8r = jax.lax.broadcasted_iota(jnp.int32, (_E, _E), 0)
    i8c = jax.lax.broadcasted_iota(jnp.int32, (_E, _E), 1)
    su8 = (i8r < i8c).astype(jnp.float32)
    po = jnp.dot(padded, su8, preferred_element_type=jnp.float32)

    po_next = po + padded
    b_iota = jax.lax.broadcasted_iota(
        jnp.int32, (64, 1), 0).astype(jnp.float32) * _BLK
    ccmp = (po_next <= b_iota).astype(jnp.int32)
    be_ref[...] = jnp.minimum(
        jnp.sum(ccmp, axis=1, keepdims=True), _E - 1)

    f = cnt / jnp.float32(_T * _K)
    pm = jnp.mean(p, axis=0, keepdims=True)
    aux = jnp.float32(_E * _COEF) * jnp.sum(f * pm)
    aux_ref[...] = jnp.broadcast_to(aux, (1, 1))

    ic_r = jax.lax.broadcasted_iota(jnp.int32, (_CH, _CH), 0)
    ic_c = jax.lax.broadcasted_iota(jnp.int32, (_CH, _CH), 1)
    sl = (ic_c < ic_r).astype(jnp.float32)
    run = jnp.zeros((1, _E), jnp.float32)
    for c in range(_T // _CH):
        lo, hi = c * _CH, (c + 1) * _CH
        ch = assign[lo:hi, :]
        ex = jnp.dot(sl, ch, preferred_element_type=jnp.float32) + run
        run = run + jnp.sum(ch, axis=0, keepdims=True)
        base = ex + po
        is1_c = is1[lo:hi, :].astype(jnp.float32)
        is2_c = is2[lo:hi, :].astype(jnp.float32)
        pos1_ref[lo:hi, :] = jnp.sum(base * is1_c, axis=1,
                                     keepdims=True).astype(jnp.int32)
        pos2_ref[lo:hi, :] = jnp.sum(base * is2_c, axis=1,
                                     keepdims=True).astype(jnp.int32)


def _router_meta(x_flat, W_router, b_router):
    return pl.pallas_call(
        _router_meta_kernel,
        out_shape=(
            jax.ShapeDtypeStruct((_T, 1), jnp.float32),
            jax.ShapeDtypeStruct((_T, 1), jnp.float32),
            jax.ShapeDtypeStruct((_T, 1), jnp.int32),
            jax.ShapeDtypeStruct((_T, 1), jnp.int32),
            jax.ShapeDtypeStruct((64, 1), jnp.int32),
            jax.ShapeDtypeStruct((1, 1), jnp.float32),
        ),
    )(x_flat, W_router, b_router.reshape(1, _E))


def _sc_dispatch_body(x_hbm, pos1_hbm, pos2_hbm, wa_hbm, wb_hbm,
                      xs_hbm, ws_hbm,
                      inv_sh, ws_sh,
                      idx1_v, idx2_v, tok_v, wa_v, wb_v,
                      zi_v, zf_v, invv, wsv, row_v, row2_v,
                      sem, wsem1, wsem2):
    cid = lax.axis_index("c")
    sid = lax.axis_index("s")
    wid = cid * _NS + sid

    stripe0 = sid * (_NPAD // _NS)
    for i in range((_NPAD // _NS) // 16):
        zi_v[pl.ds(i * 16, 16)] = jnp.zeros((16,), jnp.int32)
        zf_v[pl.ds(i * 16, 16)] = jnp.zeros((16,), jnp.float32)
    pltpu.sync_copy(zi_v, inv_sh.at[pl.ds(stripe0, _NPAD // _NS)])
    pltpu.sync_copy(zf_v, ws_sh.at[pl.ds(stripe0, _NPAD // _NS)])
    plsc.subcore_barrier()

    tok0 = sid * _TPS
    pltpu.sync_copy(pos1_hbm.at[pl.ds(tok0, _TPS)], idx1_v)
    pltpu.sync_copy(pos2_hbm.at[pl.ds(tok0, _TPS)], idx2_v)
    pltpu.sync_copy(wa_hbm.at[pl.ds(tok0, _TPS)], wa_v)
    pltpu.sync_copy(wb_hbm.at[pl.ds(tok0, _TPS)], wb_v)
    for i in range(_TPS // 16):
        tok_v[pl.ds(i * 16, 16)] = lax.iota(jnp.int32, 16) + (tok0 + i * 16)
    pltpu.sync_copy(tok_v, inv_sh.at[idx1_v])
    pltpu.sync_copy(tok_v, inv_sh.at[idx2_v])
    pltpu.sync_copy(wa_v, ws_sh.at[idx1_v])
    pltpu.sync_copy(wb_v, ws_sh.at[idx2_v])
    plsc.subcore_barrier()

    slot0 = wid * _SPW
    pltpu.sync_copy(inv_sh.at[pl.ds(slot0, _SPW)], invv)
    pltpu.sync_copy(ws_sh.at[pl.ds(slot0, _SPW)], wsv)
    pltpu.sync_copy(wsv, ws_hbm.at[pl.ds(slot0, _SPW)])
    rows = (row_v, row2_v)
    wsems = (wsem1, wsem2)
    dw = [None, None]
    for c in range(_SPW // _GCH):
        i = c % 2
        if dw[i] is not None:
            dw[i].wait()
        pltpu.async_copy(
            x_hbm.at[invv.at[pl.ds(c * _GCH, _GCH)]], rows[i], sem).wait()
        dw[i] = pltpu.async_copy(
            rows[i], xs_hbm.at[pl.ds(slot0 + c * _GCH, _GCH)], wsems[i])
    dw[0].wait()
    dw[1].wait()


def _sc_dispatch(x_flat, pos1, pos2, wa, wb):
    return pl.kernel(
        _sc_dispatch_body,
        out_type=(
            jax.ShapeDtypeStruct((_NPAD, _H), jnp.float32),
            jax.ShapeDtypeStruct((_NPAD,), jnp.float32),
        ),
        mesh=_mesh,
        scratch_types=[
            pltpu.VMEM_SHARED((_NPAD,), jnp.int32),
            pltpu.VMEM_SHARED((_NPAD,), jnp.float32),
            pltpu.VMEM((_TPS,), jnp.int32),
            pltpu.VMEM((_TPS,), jnp.int32),
            pltpu.VMEM((_TPS,), jnp.int32),
            pltpu.VMEM((_TPS,), jnp.float32),
            pltpu.VMEM((_TPS,), jnp.float32),
            pltpu.VMEM((_NPAD // _NS,), jnp.int32),
            pltpu.VMEM((_NPAD // _NS,), jnp.float32),
            pltpu.VMEM((_SPW,), jnp.int32),
            pltpu.VMEM((_SPW,), jnp.float32),
            pltpu.VMEM((_GCH, _H), jnp.float32),
            pltpu.VMEM((_GCH, _H), jnp.float32),
            pltpu.SemaphoreType.DMA,
            pltpu.SemaphoreType.DMA,
            pltpu.SemaphoreType.DMA,
        ],
    )(x_flat, pos1, pos2, wa, wb)


def _up_kernel(be_ref, xs_ref, w1_ref, b1_ref, h_ref):
    xb = xs_ref[...].astype(jnp.bfloat16)
    w1 = w1_ref[0].astype(jnp.bfloat16)
    h = jnp.dot(xb, w1, preferred_element_type=jnp.float32) + b1_ref[0]
    h_ref[...] = _gelu_tanh(h).astype(jnp.bfloat16)


def _down_kernel(be_ref, h_ref, w2_ref, b2_ref, ws_ref, y_ref):
    w2 = w2_ref[0].astype(jnp.bfloat16)
    y = jnp.dot(h_ref[...], w2, preferred_element_type=jnp.float32) + b2_ref[0]
    y_ref[...] = y * ws_ref[...]


def _ffn_grouped(be40, xs, W1, b1, W2, b2, ws2d):
    b1r = b1.reshape(_E, 1, _F)
    b2r = b2.reshape(_E, 1, _H)
    h = pl.pallas_call(
        _up_kernel,
        grid_spec=pltpu.PrefetchScalarGridSpec(
            num_scalar_prefetch=1,
            grid=(_NB,),
            in_specs=[
                pl.BlockSpec((_BLK, _H), lambda b, be: (b, 0)),
                pl.BlockSpec((1, _H, _F), lambda b, be: (be[b], 0, 0)),
                pl.BlockSpec((1, 1, _F), lambda b, be: (be[b], 0, 0)),
            ],
            out_specs=pl.BlockSpec((_BLK, _F), lambda b, be: (b, 0)),
        ),
        out_shape=jax.ShapeDtypeStruct((_NPAD, _F), jnp.bfloat16),
        compiler_params=pltpu.CompilerParams(
            dimension_semantics=("arbitrary",)),
    )(be40, xs, W1, b1r)
    y = pl.pallas_call(
        _down_kernel,
        grid_spec=pltpu.PrefetchScalarGridSpec(
            num_scalar_prefetch=1,
            grid=(_NB,),
            in_specs=[
                pl.BlockSpec((_BLK, _F), lambda b, be: (b, 0)),
                pl.BlockSpec((1, _F, _H), lambda b, be: (be[b], 0, 0)),
                pl.BlockSpec((1, 1, _H), lambda b, be: (be[b], 0, 0)),
                pl.BlockSpec((_BLK, 1), lambda b, be: (b, 0)),
            ],
            out_specs=pl.BlockSpec((_BLK, _H), lambda b, be: (b, 0)),
        ),
        out_shape=jax.ShapeDtypeStruct((_NPAD, _H), jnp.float32),
        compiler_params=pltpu.CompilerParams(
            dimension_semantics=("arbitrary",)),
    )(be40, h, W2, b2r, ws2d)
    return y


def _sc_combine_body(y_hbm, pos1_hbm, pos2_hbm, out_hbm,
                     idx1_v, idx2_v, b1_v, b2_v, sem1, sem2):
    cid = lax.axis_index("c")
    sid = lax.axis_index("s")
    wid = cid * _NS + sid
    tok0 = wid * _TPW
    pltpu.sync_copy(pos1_hbm.at[pl.ds(tok0, _TPW)], idx1_v)
    pltpu.sync_copy(pos2_hbm.at[pl.ds(tok0, _TPW)], idx2_v)
    for c in range(_TPW // _CCH):
        d1 = pltpu.async_copy(
            y_hbm.at[idx1_v.at[pl.ds(c * _CCH, _CCH)]], b1_v, sem1)
        d2 = pltpu.async_copy(
            y_hbm.at[idx2_v.at[pl.ds(c * _CCH, _CCH)]], b2_v, sem2)
        d1.wait()
        d2.wait()

        @pl.loop(0, _CCH)
        def _(i):
            for v in range(_H // 16):
                sl = pl.ds(v * 16, 16)
                b1_v[i, sl] = b1_v[i, sl] + b2_v[i, sl]

        pltpu.sync_copy(b1_v, out_hbm.at[pl.ds(tok0 + c * _CCH, _CCH)])


def _sc_combine(y, pos1, pos2):
    return pl.kernel(
        _sc_combine_body,
        out_type=jax.ShapeDtypeStruct((_T, _H), jnp.float32),
        mesh=_mesh,
        scratch_types=[
            pltpu.VMEM((_TPW,), jnp.int32),
            pltpu.VMEM((_TPW,), jnp.int32),
            pltpu.VMEM((_CCH, _H), jnp.float32),
            pltpu.VMEM((_CCH, _H), jnp.float32),
            pltpu.SemaphoreType.DMA,
            pltpu.SemaphoreType.DMA,
        ],
    )(y, pos1, pos2)


def kernel(x, W_router, b_router, W1, b1, W2, b2):
    x_flat = x.reshape(_T, _H)
    w1c, w2c, pos1c, pos2c, bec, aux = _router_meta(x_flat, W_router, b_router)
    pos1 = pos1c.reshape(_T)
    pos2 = pos2c.reshape(_T)
    be40 = bec.reshape(64)[:_NB]
    xs, ws = _sc_dispatch(x_flat, pos1, pos2,
                          w1c.reshape(_T), w2c.reshape(_T))
    y = _ffn_grouped(be40, xs, W1, b1, W2, b2, ws.reshape(_NPAD, 1))
    out = _sc_combine(y, pos1, pos2)
    return out.reshape(_B, _S, _H), aux.reshape(())


# speedup vs baseline: 1.6036x; 1.0502x over previous
import functools

import jax
import jax.numpy as jnp
from jax import lax
from jax.experimental import pallas as pl
from jax.experimental.pallas import tpu as pltpu
from jax.experimental.pallas import tpu_sc as plsc

_B, _S, _H = 2, 2048, 1024
_F = 4096
_E = 8
_K = 2
_COEF = 0.01
_T = _B * _S
_BLK = 256
_NB = 40
_NPAD = _NB * _BLK
_CH = 512
_NC, _NS = 2, 16
_NW = _NC * _NS
_TPW = _T // _NW
_TPS = _T // _NS
_SPW = _NPAD // _NW
_GCH = 40
_CCH = 16

_mesh = plsc.VectorSubcoreMesh(core_axis_name="c", subcore_axis_name="s",
                               num_cores=_NC, num_subcores=_NS)


def _gelu_tanh(x):
    c = jnp.sqrt(2.0 / jnp.pi).astype(x.dtype)
    return 0.5 * x * (1.0 + jnp.tanh(c * (x + 0.044715 * (x * x * x))))


def _router_meta_kernel(x_ref, wr_ref, br_ref,
                        w1_ref, w2_ref, pos1_ref, pos2_ref, be_ref, aux_ref):
    x = x_ref[...]
    wr = wr_ref[...]
    logits = jnp.dot(x, wr, preferred_element_type=jnp.float32) + br_ref[...]
    p = jax.nn.softmax(logits, axis=-1)
    e_iota = jax.lax.broadcasted_iota(jnp.int32, p.shape, 1)
    a1 = jnp.argmax(p, axis=-1)
    is1 = e_iota == a1[:, None]
    m1 = jnp.max(p, axis=-1, keepdims=True)
    p_mask = jnp.where(is1, -jnp.inf, p)
    a2 = jnp.argmax(p_mask, axis=-1)
    is2 = e_iota == a2[:, None]
    m2 = jnp.max(p_mask, axis=-1, keepdims=True)
    s = m1 + m2
    w1_ref[...] = m1 / s
    w2_ref[...] = m2 / s

    assign = (is1 | is2).astype(jnp.float32)
    cnt = jnp.sum(assign, axis=0, keepdims=True)
    padded = jnp.floor((cnt + jnp.float32(_BLK - 1)) / _BLK) * _BLK
    i8r = jax.lax.broadcasted_iota(jnp.int32, (_E, _E), 0)
    i8c = jax.lax.broadcasted_iota(jnp.int32, (_E, _E), 1)
    su8 = (i8r < i8c).astype(jnp.float32)
    po = jnp.dot(padded, su8, preferred_element_type=jnp.float32)

    po_next = po + padded
    b_iota = jax.lax.broadcasted_iota(
        jnp.int32, (64, 1), 0).astype(jnp.float32) * _BLK
    ccmp = (po_next <= b_iota).astype(jnp.int32)
    er = jax.lax.broadcasted_iota(jnp.int32, (1, _E), 1)
    lastne = jnp.max(jnp.where(padded > 0, er + 1, 0)) - 1
    bev = jnp.minimum(jnp.sum(ccmp, axis=1, keepdims=True), lastne)
    n_blk = jnp.sum(padded).astype(jnp.int32) // _BLK
    row64 = jax.lax.broadcasted_iota(jnp.int32, (64, 1), 0)
    be_ref[...] = jnp.where(row64 == _NB, n_blk, bev)

    f = cnt / jnp.float32(_T * _K)
    pm = jnp.mean(p, axis=0, keepdims=True)
    aux = jnp.float32(_E * _COEF) * jnp.sum(f * pm)
    aux_ref[...] = jnp.broadcast_to(aux, (1, 1))

    ic_r = jax.lax.broadcasted_iota(jnp.int32, (_CH, _CH), 0)
    ic_c = jax.lax.broadcasted_iota(jnp.int32, (_CH, _CH), 1)
    sl = (ic_c < ic_r).astype(jnp.float32)
    run = jnp.zeros((1, _E), jnp.float32)
    for c in range(_T // _CH):
        lo, hi = c * _CH, (c + 1) * _CH
        ch = assign[lo:hi, :]
        ex = jnp.dot(sl, ch, preferred_element_type=jnp.float32) + run
        run = run + jnp.sum(ch, axis=0, keepdims=True)
        base = ex + po
        is1_c = is1[lo:hi, :].astype(jnp.float32)
        is2_c = is2[lo:hi, :].astype(jnp.float32)
        pos1_ref[lo:hi, :] = jnp.sum(base * is1_c, axis=1,
                                     keepdims=True).astype(jnp.int32)
        pos2_ref[lo:hi, :] = jnp.sum(base * is2_c, axis=1,
                                     keepdims=True).astype(jnp.int32)


def _router_meta(x_flat, W_router, b_router):
    return pl.pallas_call(
        _router_meta_kernel,
        out_shape=(
            jax.ShapeDtypeStruct((_T, 1), jnp.float32),
            jax.ShapeDtypeStruct((_T, 1), jnp.float32),
            jax.ShapeDtypeStruct((_T, 1), jnp.int32),
            jax.ShapeDtypeStruct((_T, 1), jnp.int32),
            jax.ShapeDtypeStruct((64, 1), jnp.int32),
            jax.ShapeDtypeStruct((1, 1), jnp.float32),
        ),
    )(x_flat, W_router, b_router.reshape(1, _E))


def _sc_dispatch_body(x_hbm, pos1_hbm, pos2_hbm, wa_hbm, wb_hbm,
                      xs_hbm, ws_hbm,
                      inv_sh, ws_sh,
                      idx1_v, idx2_v, tok_v, wa_v, wb_v,
                      zi_v, zf_v, invv, wsv, row_v, row2_v,
                      sem, wsem1, wsem2):
    cid = lax.axis_index("c")
    sid = lax.axis_index("s")
    wid = cid * _NS + sid

    stripe0 = sid * (_NPAD // _NS)
    for i in range((_NPAD // _NS) // 16):
        zi_v[pl.ds(i * 16, 16)] = jnp.zeros((16,), jnp.int32)
        zf_v[pl.ds(i * 16, 16)] = jnp.zeros((16,), jnp.float32)
    pltpu.sync_copy(zi_v, inv_sh.at[pl.ds(stripe0, _NPAD // _NS)])
    pltpu.sync_copy(zf_v, ws_sh.at[pl.ds(stripe0, _NPAD // _NS)])
    plsc.subcore_barrier()

    tok0 = sid * _TPS
    pltpu.sync_copy(pos1_hbm.at[pl.ds(tok0, _TPS)], idx1_v)
    pltpu.sync_copy(pos2_hbm.at[pl.ds(tok0, _TPS)], idx2_v)
    pltpu.sync_copy(wa_hbm.at[pl.ds(tok0, _TPS)], wa_v)
    pltpu.sync_copy(wb_hbm.at[pl.ds(tok0, _TPS)], wb_v)
    for i in range(_TPS // 16):
        tok_v[pl.ds(i * 16, 16)] = lax.iota(jnp.int32, 16) + (tok0 + i * 16)
    pltpu.sync_copy(tok_v, inv_sh.at[idx1_v])
    pltpu.sync_copy(tok_v, inv_sh.at[idx2_v])
    pltpu.sync_copy(wa_v, ws_sh.at[idx1_v])
    pltpu.sync_copy(wb_v, ws_sh.at[idx2_v])
    plsc.subcore_barrier()

    slot0 = wid * _SPW
    pltpu.sync_copy(inv_sh.at[pl.ds(slot0, _SPW)], invv)
    pltpu.sync_copy(ws_sh.at[pl.ds(slot0, _SPW)], wsv)
    pltpu.sync_copy(wsv, ws_hbm.at[pl.ds(slot0, _SPW)])
    rows = (row_v, row2_v)
    wsems = (wsem1, wsem2)
    dw = [None, None]
    for c in range(_SPW // _GCH):
        i = c % 2
        if dw[i] is not None:
            dw[i].wait()
        pltpu.async_copy(
            x_hbm.at[invv.at[pl.ds(c * _GCH, _GCH)]], rows[i], sem).wait()
        dw[i] = pltpu.async_copy(
            rows[i], xs_hbm.at[pl.ds(slot0 + c * _GCH, _GCH)], wsems[i])
    dw[0].wait()
    dw[1].wait()


def _sc_dispatch(x_flat, pos1, pos2, wa, wb):
    return pl.kernel(
        _sc_dispatch_body,
        out_type=(
            jax.ShapeDtypeStruct((_NPAD, _H), jnp.float32),
            jax.ShapeDtypeStruct((_NPAD,), jnp.float32),
        ),
        mesh=_mesh,
        scratch_types=[
            pltpu.VMEM_SHARED((_NPAD,), jnp.int32),
            pltpu.VMEM_SHARED((_NPAD,), jnp.float32),
            pltpu.VMEM((_TPS,), jnp.int32),
            pltpu.VMEM((_TPS,), jnp.int32),
            pltpu.VMEM((_TPS,), jnp.int32),
            pltpu.VMEM((_TPS,), jnp.float32),
            pltpu.VMEM((_TPS,), jnp.float32),
            pltpu.VMEM((_NPAD // _NS,), jnp.int32),
            pltpu.VMEM((_NPAD // _NS,), jnp.float32),
            pltpu.VMEM((_SPW,), jnp.int32),
            pltpu.VMEM((_SPW,), jnp.float32),
            pltpu.VMEM((_GCH, _H), jnp.float32),
            pltpu.VMEM((_GCH, _H), jnp.float32),
            pltpu.SemaphoreType.DMA,
            pltpu.SemaphoreType.DMA,
            pltpu.SemaphoreType.DMA,
        ],
    )(x_flat, pos1, pos2, wa, wb)


def _up_kernel(be_ref, xs_ref, w1_ref, b1_ref, h_ref):
    @pl.when(pl.program_id(0) < be_ref[_NB])
    def _():
        xb = xs_ref[...].astype(jnp.bfloat16)
        w1 = w1_ref[0].astype(jnp.bfloat16)
        h = jnp.dot(xb, w1, preferred_element_type=jnp.float32) + b1_ref[0]
        h_ref[...] = _gelu_tanh(h).astype(jnp.bfloat16)


def _down_kernel(be_ref, h_ref, w2_ref, b2_ref, ws_ref, y_ref):
    @pl.when(pl.program_id(0) < be_ref[_NB])
    def _():
        w2 = w2_ref[0].astype(jnp.bfloat16)
        y = jnp.dot(h_ref[...], w2,
                    preferred_element_type=jnp.float32) + b2_ref[0]
        y_ref[...] = y * ws_ref[...]


def _ffn_grouped(be40, xs, W1, b1, W2, b2, ws2d):
    b1r = b1.reshape(_E, 1, _F)
    b2r = b2.reshape(_E, 1, _H)
    h = pl.pallas_call(
        _up_kernel,
        grid_spec=pltpu.PrefetchScalarGridSpec(
            num_scalar_prefetch=1,
            grid=(_NB,),
            in_specs=[
                pl.BlockSpec((_BLK, _H), lambda b, be: (b, 0)),
                pl.BlockSpec((1, _H, _F), lambda b, be: (be[b], 0, 0)),
                pl.BlockSpec((1, 1, _F), lambda b, be: (be[b], 0, 0)),
            ],
            out_specs=pl.BlockSpec((_BLK, _F), lambda b, be: (b, 0)),
        ),
        out_shape=jax.ShapeDtypeStruct((_NPAD, _F), jnp.bfloat16),
        compiler_params=pltpu.CompilerParams(
            dimension_semantics=("arbitrary",)),
    )(be40, xs, W1, b1r)
    y = pl.pallas_call(
        _down_kernel,
        grid_spec=pltpu.PrefetchScalarGridSpec(
            num_scalar_prefetch=1,
            grid=(_NB,),
            in_specs=[
                pl.BlockSpec((_BLK, _F), lambda b, be: (b, 0)),
                pl.BlockSpec((1, _F, _H), lambda b, be: (be[b], 0, 0)),
                pl.BlockSpec((1, 1, _H), lambda b, be: (be[b], 0, 0)),
                pl.BlockSpec((_BLK, 1), lambda b, be: (b, 0)),
            ],
            out_specs=pl.BlockSpec((_BLK, _H), lambda b, be: (b, 0)),
        ),
        out_shape=jax.ShapeDtypeStruct((_NPAD, _H), jnp.float32),
        compiler_params=pltpu.CompilerParams(
            dimension_semantics=("arbitrary",)),
    )(be40, h, W2, b2r, ws2d)
    return y


def _sc_combine_body(y_hbm, pos1_hbm, pos2_hbm, out_hbm,
                     idx1_v, idx2_v, b1a_v, b1b_v, b2a_v, b2b_v,
                     g1a, g1b, g2a, g2b, wsa, wsb):
    cid = lax.axis_index("c")
    sid = lax.axis_index("s")
    wid = cid * _NS + sid
    tok0 = wid * _TPW
    pltpu.sync_copy(pos1_hbm.at[pl.ds(tok0, _TPW)], idx1_v)
    pltpu.sync_copy(pos2_hbm.at[pl.ds(tok0, _TPW)], idx2_v)
    b1s, b2s = (b1a_v, b1b_v), (b2a_v, b2b_v)
    g1s, g2s, wss = (g1a, g1b), (g2a, g2b), (wsa, wsb)
    nch = _TPW // _CCH
    dg1 = [None, None]
    dg2 = [None, None]
    dw = [None, None]

    def fire(c):
        i = c % 2
        dg1[i] = pltpu.async_copy(
            y_hbm.at[idx1_v.at[pl.ds(c * _CCH, _CCH)]], b1s[i], g1s[i])
        dg2[i] = pltpu.async_copy(
            y_hbm.at[idx2_v.at[pl.ds(c * _CCH, _CCH)]], b2s[i], g2s[i])

    fire(0)
    for c in range(nch):
        i = c % 2
        dg1[i].wait()
        dg2[i].wait()
        if c + 1 < nch:
            j = (c + 1) % 2
            if dw[j] is not None:
                dw[j].wait()
            fire(c + 1)

        @pl.loop(0, _CCH)
        def _(r):
            for v in range(_H // 16):
                sl = pl.ds(v * 16, 16)
                b1s[i][r, sl] = b1s[i][r, sl] + b2s[i][r, sl]

        dw[i] = pltpu.async_copy(
            b1s[i], out_hbm.at[pl.ds(tok0 + c * _CCH, _CCH)], wss[i])
    dw[0].wait()
    dw[1].wait()


def _sc_combine(y, pos1, pos2):
    return pl.kernel(
        _sc_combine_body,
        out_type=jax.ShapeDtypeStruct((_T, _H), jnp.float32),
        mesh=_mesh,
        scratch_types=[
            pltpu.VMEM((_TPW,), jnp.int32),
            pltpu.VMEM((_TPW,), jnp.int32),
            pltpu.VMEM((_CCH, _H), jnp.float32),
            pltpu.VMEM((_CCH, _H), jnp.float32),
            pltpu.VMEM((_CCH, _H), jnp.float32),
            pltpu.VMEM((_CCH, _H), jnp.float32),
            pltpu.SemaphoreType.DMA,
            pltpu.SemaphoreType.DMA,
            pltpu.SemaphoreType.DMA,
            pltpu.SemaphoreType.DMA,
            pltpu.SemaphoreType.DMA,
            pltpu.SemaphoreType.DMA,
        ],
    )(y, pos1, pos2)


def kernel(x, W_router, b_router, W1, b1, W2, b2):
    x_flat = x.reshape(_T, _H)
    w1c, w2c, pos1c, pos2c, bec, aux = _router_meta(x_flat, W_router, b_router)
    pos1 = pos1c.reshape(_T)
    pos2 = pos2c.reshape(_T)
    be40 = bec.reshape(64)[:_NB + 1]
    xs, ws = _sc_dispatch(x_flat, pos1, pos2,
                          w1c.reshape(_T), w2c.reshape(_T))
    y = _ffn_grouped(be40, xs, W1, b1, W2, b2, ws.reshape(_NPAD, 1))
    out = _sc_combine(y, pos1, pos2)
    return out.reshape(_B, _S, _H), aux.reshape(())
